# baseline (speedup 1.0000x reference)
import functools

import jax
import jax.numpy as jnp
from jax import lax
from jax.experimental import pallas as pl
from jax.experimental.pallas import tpu as pltpu
from jax.experimental.pallas import tpu_sc as plsc

B = 8
S = 20
T = B * S
NDET = 1024
NANCH = 16
NOUT = 64
NLOOKUP = NDET * NANCH

NC = 2
NS = 16
L = 16

BPC = B // NC
TPC = T // NC
DPS = NDET // NS
TG = TPC // L


def _body(x_hbm, w_hbm, anch_hbm, ids_hbm,
          out_hbm, lut_hbm, mind_hbm, amin_hbm,
          x_v, w_v, anch_v, ids_v, lut_v, mind_v, amin_v, acc_v, tidx_v,
          acc_sh):
    cid = lax.axis_index("c")
    sid = lax.axis_index("s")
    b0 = cid * BPC
    d0 = sid * DPS

    pltpu.sync_copy(x_hbm.at[pl.ds(b0, BPC)], x_v)
    pltpu.sync_copy(w_hbm.at[pl.ds(d0 * NANCH, DPS * NANCH), :], w_v)
    pltpu.sync_copy(anch_hbm.at[pl.ds(d0, DPS), :], anch_v)
    pltpu.sync_copy(ids_hbm.at[pl.ds(d0, DPS)], ids_v)

    iota = lax.iota(jnp.int32, L)
    for g in range(TPC // L):
        tidx_v[pl.ds(g * L, L)] = iota + g * L

    def d_body(d, carry):
        di = jnp.full((L,), d, jnp.int32)
        fi = plsc.load_gather(ids_v, [di])
        gd16 = (d0 + d) * NANCH
        arow = anch_v[d, :]
        anchs = [jnp.full((L,), arow[a]) for a in range(NANCH)]
        for tg in range(TG):
            ti = iota + tg * L
            bi = ti // S
            si = ti - bi * S
            xi = plsc.load_gather(x_v, [bi, si, fi])
            best = jnp.abs(xi - anchs[0])
            besta = jnp.zeros((L,), jnp.int32)
            for a in range(1, NANCH):
                dl = jnp.abs(xi - anchs[a])
                m = dl < best
                besta = jnp.where(m, a, besta)
                best = jnp.where(m, dl, best)
            plsc.store_scatter(mind_v, [bi, si, di], best)
            plsc.store_scatter(amin_v, [bi, si, di], besta)
            plsc.store_scatter(lut_v, [bi, si, di], gd16 + besta)
        return carry

    lax.fori_loop(0, DPS, d_body, 0)

    pltpu.sync_copy(lut_v, lut_hbm.at[pl.ds(b0, BPC), :, pl.ds(d0, DPS)])
    pltpu.sync_copy(mind_v, mind_hbm.at[pl.ds(b0, BPC), :, pl.ds(d0, DPS)])
    pltpu.sync_copy(amin_v, amin_hbm.at[pl.ds(b0, BPC), :, pl.ds(d0, DPS)])

    zero = jnp.zeros((L,), jnp.float32)

    def t_body(t, carry):
        tb = t // S
        ts_ = t - tb * S

        def dd_body(dg, accs):
            a0, a1, a2, a3 = accs
            amv = amin_v[tb, ts_, pl.ds(dg * L, L)]
            cv = 1.0 - mind_v[tb, ts_, pl.ds(dg * L, L)]
            for u in range(L):
                r = (dg * L + u) * NANCH + amv[u]
                c = cv[u]
                a0 = a0 + c * w_v[r, pl.ds(0, L)]
                a1 = a1 + c * w_v[r, pl.ds(L, L)]
                a2 = a2 + c * w_v[r, pl.ds(2 * L, L)]
                a3 = a3 + c * w_v[r, pl.ds(3 * L, L)]
            return (a0, a1, a2, a3)

        a0, a1, a2, a3 = lax.fori_loop(0, DPS // L, dd_body,
                                       (zero, zero, zero, zero))
        acc_v[t, pl.ds(0, L)] = a0
        acc_v[t, pl.ds(L, L)] = a1
        acc_v[t, pl.ds(2 * L, L)] = a2
        acc_v[t, pl.ds(3 * L, L)] = a3
        return carry

    lax.fori_loop(0, TPC, t_body, 0)

    plsc.subcore_barrier()

    @pl.when(sid == 0)
    def _():
        pltpu.sync_copy(acc_v, acc_sh)

    plsc.subcore_barrier()

    @pl.when(sid != 0)
    def _():
        pltpu.sync_copy(acc_v, acc_sh.at[tidx_v], add=True)

    plsc.subcore_barrier()

    @pl.when(sid == 0)
    def _():
        for i in range(BPC):
            pltpu.sync_copy(acc_sh.at[pl.ds(i * S, S), :], out_hbm.at[b0 + i])


_lut_sc = functools.partial(
    pl.kernel,
    out_type=(
        jax.ShapeDtypeStruct((B, S, NOUT), jnp.float32),
        jax.ShapeDtypeStruct((B, S, NDET), jnp.int32),
        jax.ShapeDtypeStruct((B, S, NDET), jnp.float32),
        jax.ShapeDtypeStruct((B, S, NDET), jnp.int32),
    ),
    mesh=plsc.VectorSubcoreMesh(core_axis_name="c", subcore_axis_name="s",
                                num_cores=NC, num_subcores=NS),
    compiler_params=pltpu.CompilerParams(use_tc_tiling_on_sc=False,
                                         needs_layout_passes=False),
    scratch_types=[
        pltpu.VMEM((BPC, S, NOUT), jnp.float32),
        pltpu.VMEM((DPS * NANCH, NOUT), jnp.float32),
        pltpu.VMEM((DPS, NANCH), jnp.float32),
        pltpu.VMEM((DPS,), jnp.int32),
        pltpu.VMEM((BPC, S, DPS), jnp.int32),
        pltpu.VMEM((BPC, S, DPS), jnp.float32),
        pltpu.VMEM((BPC, S, DPS), jnp.int32),
        pltpu.VMEM((TPC, NOUT), jnp.float32),
        pltpu.VMEM((TPC,), jnp.int32),
        pltpu.VMEM_SHARED((TPC, NOUT), jnp.float32),
    ],
)(_body)


@jax.jit
def kernel(x, weights, anchors, detector_input_ids):
    return _lut_sc(x, weights, anchors, detector_input_ids)


# speedup vs baseline: 1.3026x; 1.3026x over previous
import functools

import jax
import jax.numpy as jnp
from jax import lax
from jax.experimental import pallas as pl
from jax.experimental.pallas import tpu as pltpu
from jax.experimental.pallas import tpu_sc as plsc

B = 8
S = 20
T = B * S
NDET = 1024
NANCH = 16
NOUT = 64
NLOOKUP = NDET * NANCH

NC = 2
NS = 16
L = 16

BPC = B // NC
TPC = T // NC
DPS = NDET // NS
TG = TPC // L
NOP = NOUT + 1
DPP = DPS + 1


def _body_a(x_hbm, anch_hbm, ids_hbm,
            lut_hbm, mind_hbm, amin_hbm,
            x_v, anch_v, ids_v, lut_v, mind_v, amin_v):
    cid = lax.axis_index("c")
    sid = lax.axis_index("s")
    b0 = cid * BPC
    d0 = sid * DPS

    ns = jax.named_scope
    with ns("a_in"):
        pltpu.sync_copy(x_hbm.at[pl.ds(b0, BPC)],
                        x_v.at[:, :, pl.ds(0, NOUT)])
        pltpu.sync_copy(anch_hbm.at[pl.ds(d0, DPS), :], anch_v)
        pltpu.sync_copy(ids_hbm.at[pl.ds(d0, DPS)], ids_v)

    iota = lax.iota(jnp.int32, L)

    def d_body(d, carry):
        di = jnp.full((L,), d, jnp.int32)
        fi = plsc.load_gather(ids_v, [di])
        gd16 = (d0 + d) * NANCH
        arow = anch_v[d, :]
        anchs = [jnp.full((L,), arow[a]) for a in range(NANCH)]
        for tg in range(TG):
            ti = iota + tg * L
            bi = ti // S
            si = ti - bi * S
            xi = plsc.load_gather(x_v, [bi, si, fi])
            pairs = [(jnp.abs(xi - anchs[a]), jnp.full((L,), a, jnp.int32))
                     for a in range(NANCH)]
            while len(pairs) > 1:
                nxt = []
                for k in range(0, len(pairs), 2):
                    (va, ia), (vb, ib) = pairs[k], pairs[k + 1]
                    m = vb < va
                    nxt.append((jnp.where(m, vb, va), jnp.where(m, ib, ia)))
                pairs = nxt
            best, besta = pairs[0]
            plsc.store_scatter(mind_v, [bi, si, di], best)
            plsc.store_scatter(amin_v, [bi, si, di], besta)
            plsc.store_scatter(lut_v, [bi, si, di], gd16 + besta)
        return carry

    with ns("phase1"):
        lax.fori_loop(0, DPS, d_body, 0)

    with ns("a_out"):
        pltpu.sync_copy(lut_v.at[:, :, pl.ds(0, DPS)],
                        lut_hbm.at[pl.ds(b0, BPC), :, pl.ds(d0, DPS)])
        pltpu.sync_copy(mind_v.at[:, :, pl.ds(0, DPS)],
                        mind_hbm.at[pl.ds(b0, BPC), :, pl.ds(d0, DPS)])
        pltpu.sync_copy(amin_v.at[:, :, pl.ds(0, DPS)],
                        amin_hbm.at[pl.ds(b0, BPC), :, pl.ds(d0, DPS)])


def _body_b(w_hbm, mind_hbm, amin_hbm,
            out_hbm,
            w_v, mind_v, amin_v, acc_v, tidx_v, acc_sh):
    cid = lax.axis_index("c")
    sid = lax.axis_index("s")
    b0 = cid * BPC
    d0 = sid * DPS

    ns = jax.named_scope
    with ns("b_in"):
        pltpu.sync_copy(w_hbm.at[pl.ds(d0 * NANCH, DPS * NANCH), :], w_v)
        pltpu.sync_copy(mind_hbm.at[pl.ds(b0, BPC), :, pl.ds(d0, DPS)],
                        mind_v)
        pltpu.sync_copy(amin_hbm.at[pl.ds(b0, BPC), :, pl.ds(d0, DPS)],
                        amin_v)

    iota = lax.iota(jnp.int32, L)
    for g in range(TPC // L):
        tidx_v[pl.ds(g * L, L)] = iota + g * L

    zero = jnp.zeros((L,), jnp.float32)

    def t_body(t, carry):
        tb = t // S
        ts_ = t - tb * S
        a0, a1, a2, a3 = zero, zero, zero, zero
        for dg in range(DPS // L):
            amv = amin_v[tb, ts_, pl.ds(dg * L, L)]
            cv = 1.0 - mind_v[tb, ts_, pl.ds(dg * L, L)]
            for u in range(L):
                r = (dg * L + u) * NANCH + amv[u]
                c = cv[u]
                a0 = a0 + c * w_v[r, pl.ds(0, L)]
                a1 = a1 + c * w_v[r, pl.ds(L, L)]
                a2 = a2 + c * w_v[r, pl.ds(2 * L, L)]
                a3 = a3 + c * w_v[r, pl.ds(3 * L, L)]
        acc_v[t, pl.ds(0, L)] = a0
        acc_v[t, pl.ds(L, L)] = a1
        acc_v[t, pl.ds(2 * L, L)] = a2
        acc_v[t, pl.ds(3 * L, L)] = a3
        return carry

    with ns("phase2"):
        lax.fori_loop(0, TPC, t_body, 0)

    with ns("reduce"):
        plsc.subcore_barrier()

        @pl.when(sid == 0)
        def _():
            pltpu.sync_copy(acc_v, acc_sh)

        plsc.subcore_barrier()

        @pl.when(sid != 0)
        def _():
            pltpu.sync_copy(acc_v, acc_sh.at[tidx_v], add=True)

        plsc.subcore_barrier()

        @pl.when(sid == 0)
        def _():
            for i in range(BPC):
                pltpu.sync_copy(acc_sh.at[pl.ds(i * S, S), :],
                                out_hbm.at[b0 + i])


_mesh = plsc.VectorSubcoreMesh(core_axis_name="c", subcore_axis_name="s",
                               num_cores=NC, num_subcores=NS)
_params = pltpu.CompilerParams(use_tc_tiling_on_sc=False,
                               needs_layout_passes=False)

_route_sc = functools.partial(
    pl.kernel,
    out_type=(
        jax.ShapeDtypeStruct((B, S, NDET), jnp.int32),
        jax.ShapeDtypeStruct((B, S, NDET), jnp.float32),
        jax.ShapeDtypeStruct((B, S, NDET), jnp.int32),
    ),
    mesh=_mesh,
    compiler_params=_params,
    scratch_types=[
        pltpu.VMEM((BPC, S, NOP), jnp.float32),
        pltpu.VMEM((DPS, NANCH), jnp.float32),
        pltpu.VMEM((DPS,), jnp.int32),
        pltpu.VMEM((BPC, S, DPP), jnp.int32),
        pltpu.VMEM((BPC, S, DPP), jnp.float32),
        pltpu.VMEM((BPC, S, DPP), jnp.int32),
    ],
)(_body_a)

_accum_sc = functools.partial(
    pl.kernel,
    out_type=jax.ShapeDtypeStruct((B, S, NOUT), jnp.float32),
    mesh=_mesh,
    compiler_params=_params,
    scratch_types=[
        pltpu.VMEM((DPS * NANCH, NOUT), jnp.float32),
        pltpu.VMEM((BPC, S, DPS), jnp.float32),
        pltpu.VMEM((BPC, S, DPS), jnp.int32),
        pltpu.VMEM((TPC, NOUT), jnp.float32),
        pltpu.VMEM((TPC,), jnp.int32),
        pltpu.VMEM_SHARED((TPC, NOUT), jnp.float32),
    ],
)(_body_b)


@jax.jit
def kernel(x, weights, anchors, detector_input_ids):
    lut, mind, amin = _route_sc(x, anchors, detector_input_ids)
    out = _accum_sc(weights, mind, amin)
    return (out, lut, mind, amin)


# speedup vs baseline: 1.3272x; 1.0189x over previous
import functools

import jax
import jax.numpy as jnp
from jax import lax
from jax.experimental import pallas as pl
from jax.experimental.pallas import tpu as pltpu
from jax.experimental.pallas import tpu_sc as plsc

B = 8
S = 20
T = B * S
NDET = 1024
NANCH = 16
NOUT = 64
NLOOKUP = NDET * NANCH

NC = 2
NS = 16
L = 16

BPC = B // NC
TPC = T // NC
DPS = NDET // NS
TG = TPC // L
NOP = NOUT + 1
DPP = DPS + 1


def _body_a(x_hbm, anch_hbm, ids_hbm,
            lut_hbm, mind_hbm, amin_hbm,
            x_v, anch_v, ids_v, lut_v, mind_v, amin_v):
    cid = lax.axis_index("c")
    sid = lax.axis_index("s")
    b0 = cid * BPC
    d0 = sid * DPS

    ns = jax.named_scope
    with ns("a_in"):
        pltpu.sync_copy(x_hbm.at[pl.ds(b0, BPC)],
                        x_v.at[:, :, pl.ds(0, NOUT)])
        pltpu.sync_copy(anch_hbm.at[pl.ds(d0, DPS), :], anch_v)
        pltpu.sync_copy(ids_hbm.at[pl.ds(d0, DPS)], ids_v)

    iota = lax.iota(jnp.int32, L)

    def d_body(d, carry):
        di = jnp.full((L,), d, jnp.int32)
        fi = plsc.load_gather(ids_v, [di])
        gd16 = (d0 + d) * NANCH
        arow = anch_v[d, :]
        anchs = [jnp.full((L,), arow[a]) for a in range(NANCH)]
        for tg in range(TG):
            ti = iota + tg * L
            bi = ti // S
            si = ti - bi * S
            xi = plsc.load_gather(x_v, [bi, si, fi])
            pairs = [(jnp.abs(xi - anchs[a]), jnp.full((L,), a, jnp.int32))
                     for a in range(NANCH)]
            while len(pairs) > 1:
                nxt = []
                for k in range(0, len(pairs), 2):
                    (va, ia), (vb, ib) = pairs[k], pairs[k + 1]
                    m = vb < va
                    nxt.append((jnp.where(m, vb, va), jnp.where(m, ib, ia)))
                pairs = nxt
            best, besta = pairs[0]
            plsc.store_scatter(mind_v, [bi, si, di], best)
            plsc.store_scatter(amin_v, [bi, si, di], besta)
            plsc.store_scatter(lut_v, [bi, si, di], gd16 + besta)
        return carry

    with ns("phase1"):
        lax.fori_loop(0, DPS, d_body, 0)

    with ns("a_out"):
        pltpu.sync_copy(lut_v.at[:, :, pl.ds(0, DPS)],
                        lut_hbm.at[pl.ds(b0, BPC), :, pl.ds(d0, DPS)])
        pltpu.sync_copy(mind_v.at[:, :, pl.ds(0, DPS)],
                        mind_hbm.at[pl.ds(b0, BPC), :, pl.ds(d0, DPS)])
        pltpu.sync_copy(amin_v.at[:, :, pl.ds(0, DPS)],
                        amin_hbm.at[pl.ds(b0, BPC), :, pl.ds(d0, DPS)])


def _body_b(w_hbm, mind_hbm, amin_hbm,
            out_hbm,
            w_v, mind_v, amin_v, acc_v, tidx_v, acc_sh, sem):
    cid = lax.axis_index("c")
    sid = lax.axis_index("s")
    b0 = cid * BPC
    d0 = sid * DPS

    ns = jax.named_scope
    with ns("b_in"):
        descs = [
            pltpu.async_copy(w_hbm.at[pl.ds(d0 * NANCH, DPS * NANCH), :],
                             w_v, sem),
            pltpu.async_copy(mind_hbm.at[pl.ds(b0, BPC), :, pl.ds(d0, DPS)],
                             mind_v, sem),
            pltpu.async_copy(amin_hbm.at[pl.ds(b0, BPC), :, pl.ds(d0, DPS)],
                             amin_v, sem),
        ]
        for dsc in descs:
            dsc.wait()

    iota = lax.iota(jnp.int32, L)
    for g in range(TPC // L):
        tidx_v[pl.ds(g * L, L)] = iota + g * L

    zero = jnp.zeros((L,), jnp.float32)

    def t_body(t, carry):
        tb = t // S
        ts_ = t - tb * S
        a0, a1, a2, a3 = zero, zero, zero, zero
        for dg in range(DPS // L):
            amv = amin_v[tb, ts_, pl.ds(dg * L, L)]
            cv = 1.0 - mind_v[tb, ts_, pl.ds(dg * L, L)]
            for u in range(L):
                r = (dg * L + u) * NANCH + amv[u]
                c = cv[u]
                a0 = a0 + c * w_v[r, pl.ds(0, L)]
                a1 = a1 + c * w_v[r, pl.ds(L, L)]
                a2 = a2 + c * w_v[r, pl.ds(2 * L, L)]
                a3 = a3 + c * w_v[r, pl.ds(3 * L, L)]
        acc_v[t, pl.ds(0, L)] = a0
        acc_v[t, pl.ds(L, L)] = a1
        acc_v[t, pl.ds(2 * L, L)] = a2
        acc_v[t, pl.ds(3 * L, L)] = a3
        return carry

    with ns("phase2"):
        lax.fori_loop(0, TPC, t_body, 0)

    with ns("reduce"):
        plsc.subcore_barrier()

        @pl.when(sid == 0)
        def _():
            pltpu.sync_copy(acc_v, acc_sh)

        plsc.subcore_barrier()

        @pl.when(sid != 0)
        def _():
            pltpu.sync_copy(acc_v, acc_sh.at[tidx_v], add=True)

        plsc.subcore_barrier()

        @pl.when(sid == 0)
        def _():
            for i in range(BPC):
                pltpu.sync_copy(acc_sh.at[pl.ds(i * S, S), :],
                                out_hbm.at[b0 + i])


_mesh = plsc.VectorSubcoreMesh(core_axis_name="c", subcore_axis_name="s",
                               num_cores=NC, num_subcores=NS)
_params = pltpu.CompilerParams(use_tc_tiling_on_sc=False,
                               needs_layout_passes=False)

_route_sc = functools.partial(
    pl.kernel,
    out_type=(
        jax.ShapeDtypeStruct((B, S, NDET), jnp.int32),
        jax.ShapeDtypeStruct((B, S, NDET), jnp.float32),
        jax.ShapeDtypeStruct((B, S, NDET), jnp.int32),
    ),
    mesh=_mesh,
    compiler_params=_params,
    scratch_types=[
        pltpu.VMEM((BPC, S, NOP), jnp.float32),
        pltpu.VMEM((DPS, NANCH), jnp.float32),
        pltpu.VMEM((DPS,), jnp.int32),
        pltpu.VMEM((BPC, S, DPP), jnp.int32),
        pltpu.VMEM((BPC, S, DPP), jnp.float32),
        pltpu.VMEM((BPC, S, DPP), jnp.int32),
    ],
)(_body_a)

_accum_sc = functools.partial(
    pl.kernel,
    out_type=jax.ShapeDtypeStruct((B, S, NOUT), jnp.float32),
    mesh=_mesh,
    compiler_params=_params,
    scratch_types=[
        pltpu.VMEM((DPS * NANCH, NOUT), jnp.float32),
        pltpu.VMEM((BPC, S, DPS), jnp.float32),
        pltpu.VMEM((BPC, S, DPS), jnp.int32),
        pltpu.VMEM((TPC, NOUT), jnp.float32),
        pltpu.VMEM((TPC,), jnp.int32),
        pltpu.VMEM_SHARED((TPC, NOUT), jnp.float32),
        pltpu.SemaphoreType.DMA,
    ],
)(_body_b)


@jax.jit
def kernel(x, weights, anchors, detector_input_ids):
    lut, mind, amin = _route_sc(x, anchors, detector_input_ids)
    out = _accum_sc(weights, mind, amin)
    return (out, lut, mind, amin)
